# Initial kernel scaffold; baseline (speedup 1.0000x reference)
import jax
import jax.numpy as jnp
from jax.experimental import pallas as pl


def kernel(x, table):
    raise NotImplementedError("write your pallas kernel here")


# speedup vs baseline: 5.4175x; 5.4175x over previous
import functools
import math

import jax
import jax.numpy as jnp
from jax import lax
from jax.experimental import pallas as pl
from jax.experimental.pallas import tpu as pltpu
from jax.experimental.pallas import tpu_sc as plsc

DIM = 128
SCALE = math.sqrt(DIM)

NUM_CORES = 2
NUM_SUBCORES = 16
NUM_WORKERS = NUM_CORES * NUM_SUBCORES
LANES = 16

CHUNK = 256
IDX_PER_STREAM = 128


def _embed_kernel(B):
    b_per_w = B // NUM_WORKERS
    n_chunks = b_per_w // CHUNK
    mesh = plsc.VectorSubcoreMesh(
        core_axis_name="c", subcore_axis_name="s",
        num_cores=NUM_CORES, num_subcores=NUM_SUBCORES)

    @functools.partial(
        pl.kernel,
        mesh=mesh,
        out_type=jax.ShapeDtypeStruct((B, DIM), jnp.float32),
        scratch_types=[
            pltpu.VMEM((CHUNK,), jnp.int32),
            pltpu.VMEM((CHUNK, DIM), jnp.float32),
            pltpu.SemaphoreType.DMA,
        ],
    )
    def k(x_hbm, table_hbm, out_hbm, idx_v, rows_v, sem):
        wid = lax.axis_index("s") * NUM_CORES + lax.axis_index("c")
        base = wid * b_per_w

        def chunk_body(ci, carry):
            off = base + ci * CHUNK
            pltpu.sync_copy(x_hbm.at[pl.ds(off, CHUNK)], idx_v)
            copies = []
            for j in range(CHUNK // IDX_PER_STREAM):
                s = j * IDX_PER_STREAM
                copies.append(pltpu.async_copy(
                    table_hbm.at[idx_v.at[pl.ds(s, IDX_PER_STREAM)]],
                    rows_v.at[pl.ds(s, IDX_PER_STREAM)],
                    sem))
            for cp in copies:
                cp.wait()

            def scale_row(r, c):
                for d in range(DIM // LANES):
                    sl = pl.ds(d * LANES, LANES)
                    rows_v[r, sl] = rows_v[r, sl] * SCALE
                return c

            lax.fori_loop(0, CHUNK, scale_row, 0)
            pltpu.sync_copy(rows_v, out_hbm.at[pl.ds(off, CHUNK)])
            return carry

        lax.fori_loop(0, n_chunks, chunk_body, 0)

    return k


def kernel(x, table):
    S, T = x.shape
    B = S * T
    flat = x.reshape(B).astype(jnp.int32)
    out = _embed_kernel(B)(flat, table)
    return out.reshape(S, T, DIM)


# speedup vs baseline: 9.1978x; 1.6978x over previous
import functools
import math

import jax
import jax.numpy as jnp
from jax import lax
from jax.experimental import pallas as pl
from jax.experimental.pallas import tpu as pltpu
from jax.experimental.pallas import tpu_sc as plsc

DIM = 128
SCALE = math.sqrt(DIM)

NUM_CORES = 2
NUM_SUBCORES = 16
NUM_WORKERS = NUM_CORES * NUM_SUBCORES
LANES = 16

CHUNK = 256
IDX_PER_STREAM = 128
STREAMS = CHUNK // IDX_PER_STREAM


def _embed_kernel(B):
    b_per_w = B // NUM_WORKERS
    n_chunks = b_per_w // CHUNK
    assert n_chunks % 2 == 0 and n_chunks >= 4
    mesh = plsc.VectorSubcoreMesh(
        core_axis_name="c", subcore_axis_name="s",
        num_cores=NUM_CORES, num_subcores=NUM_SUBCORES)

    @functools.partial(
        pl.kernel,
        mesh=mesh,
        out_type=jax.ShapeDtypeStruct((B, DIM), jnp.float32),
        scratch_types=[
            pltpu.VMEM((b_per_w,), jnp.int32),
            pltpu.VMEM((2, CHUNK, DIM), jnp.float32),
            pltpu.SemaphoreType.DMA,
            pltpu.SemaphoreType.DMA,
            pltpu.SemaphoreType.DMA,
            pltpu.SemaphoreType.DMA,
        ],
    )
    def k(x_hbm, table_hbm, out_hbm, idx_v, rows_v, gs0, gs1, ss0, ss1):
        wid = lax.axis_index("s") * NUM_CORES + lax.axis_index("c")
        base = wid * b_per_w
        gsems = (gs0, gs1)
        ssems = (ss0, ss1)

        pltpu.sync_copy(x_hbm.at[pl.ds(base, b_per_w)], idx_v)

        def start_gather(ci, b):
            for j in range(STREAMS):
                off = pl.multiple_of(ci * CHUNK + j * IDX_PER_STREAM,
                                     IDX_PER_STREAM)
                pltpu.async_copy(
                    table_hbm.at[idx_v.at[pl.ds(off, IDX_PER_STREAM)]],
                    rows_v.at[b, pl.ds(j * IDX_PER_STREAM, IDX_PER_STREAM)],
                    gsems[b])

        def wait_gather(b):
            pltpu.make_async_copy(
                table_hbm.at[pl.ds(0, CHUNK)], rows_v.at[b],
                gsems[b]).wait()

        def start_scatter(ci, b):
            off = pl.multiple_of(base + ci * CHUNK, CHUNK)
            pltpu.async_copy(rows_v.at[b], out_hbm.at[pl.ds(off, CHUNK)],
                             ssems[b])

        def wait_scatter(b):
            pltpu.make_async_copy(
                rows_v.at[b], out_hbm.at[pl.ds(0, CHUNK)], ssems[b]).wait()

        def scale(b):
            @plsc.parallel_loop(0, CHUNK, 2)
            def _(r):
                for u in range(2):
                    for d in range(DIM // LANES):
                        sl = pl.ds(d * LANES, LANES)
                        rows_v[b, r + u, sl] = rows_v[b, r + u, sl] * SCALE

        start_gather(0, 0)
        wait_gather(0)
        start_gather(1, 1)
        scale(0)
        start_scatter(0, 0)

        def pair(s, c):
            for b, delta in ((1, 1), (0, 2)):
                ci = 2 * s + delta
                nb = 1 - b
                wait_gather(b)
                wait_scatter(nb)
                start_gather(ci + 1, nb)
                scale(b)
                start_scatter(ci, b)
            return c

        lax.fori_loop(0, (n_chunks - 2) // 2, pair, 0)

        wait_gather(1)
        scale(1)
        start_scatter(n_chunks - 1, 1)
        wait_scatter(0)
        wait_scatter(1)

    return k


def kernel(x, table):
    S, T = x.shape
    B = S * T
    flat = x.reshape(B).astype(jnp.int32)
    out = _embed_kernel(B)(flat, table)
    return out.reshape(S, T, DIM)


# speedup vs baseline: 9.2252x; 1.0030x over previous
import functools
import math

import jax
import jax.numpy as jnp
from jax import lax
from jax.experimental import pallas as pl
from jax.experimental.pallas import tpu as pltpu
from jax.experimental.pallas import tpu_sc as plsc

DIM = 128
SCALE = math.sqrt(DIM)

NUM_CORES = 2
NUM_SUBCORES = 16
NUM_WORKERS = NUM_CORES * NUM_SUBCORES
LANES = 16

CHUNK = 128
NBUF = 4
LOOKAHEAD = 2


def _embed_kernel(B):
    b_per_w = B // NUM_WORKERS
    n_chunks = b_per_w // CHUNK
    assert (n_chunks - LOOKAHEAD - (NBUF - LOOKAHEAD)) % NBUF == 0
    mesh = plsc.VectorSubcoreMesh(
        core_axis_name="c", subcore_axis_name="s",
        num_cores=NUM_CORES, num_subcores=NUM_SUBCORES)

    @functools.partial(
        pl.kernel,
        mesh=mesh,
        out_type=jax.ShapeDtypeStruct((B, DIM), jnp.float32),
        scratch_types=[
            pltpu.VMEM((b_per_w,), jnp.int32),
            pltpu.VMEM((NBUF, CHUNK, DIM), jnp.float32),
            pltpu.SemaphoreType.DMA,
            pltpu.SemaphoreType.DMA,
            pltpu.SemaphoreType.DMA,
            pltpu.SemaphoreType.DMA,
            pltpu.SemaphoreType.DMA,
            pltpu.SemaphoreType.DMA,
            pltpu.SemaphoreType.DMA,
            pltpu.SemaphoreType.DMA,
        ],
    )
    def k(x_hbm, table_hbm, out_hbm, idx_v, rows_v,
          g0, g1, g2, g3, s0, s1, s2, s3):
        gsems = (g0, g1, g2, g3)
        ssems = (s0, s1, s2, s3)
        wid = lax.axis_index("s") * NUM_CORES + lax.axis_index("c")
        base = wid * b_per_w

        pltpu.sync_copy(x_hbm.at[pl.ds(base, b_per_w)], idx_v)

        def start_gather(ci, b):
            off = pl.multiple_of(ci * CHUNK, CHUNK)
            pltpu.async_copy(
                table_hbm.at[idx_v.at[pl.ds(off, CHUNK)]],
                rows_v.at[b], gsems[b])

        def wait_gather(b):
            pltpu.make_async_copy(
                table_hbm.at[pl.ds(0, CHUNK)], rows_v.at[b],
                gsems[b]).wait()

        def start_scatter(ci, b):
            off = pl.multiple_of(base + ci * CHUNK, CHUNK)
            pltpu.async_copy(rows_v.at[b], out_hbm.at[pl.ds(off, CHUNK)],
                             ssems[b])

        def wait_scatter(b):
            pltpu.make_async_copy(
                rows_v.at[b], out_hbm.at[pl.ds(0, CHUNK)], ssems[b]).wait()

        def scale(b):
            @plsc.parallel_loop(0, CHUNK, 2)
            def _(r):
                for u in range(2):
                    for d in range(DIM // LANES):
                        sl = pl.ds(d * LANES, LANES)
                        rows_v[b, r + u, sl] = rows_v[b, r + u, sl] * SCALE

        for j in range(LOOKAHEAD):
            start_gather(j, j)
        for ci in range(LOOKAHEAD):
            b = ci % NBUF
            wait_gather(b)
            scale(b)
            start_scatter(ci, b)
            start_gather(ci + LOOKAHEAD, (ci + LOOKAHEAD) % NBUF)

        def ring_step(s, c):
            for r in range(NBUF):
                cb = LOOKAHEAD + r
                ci = s * NBUF + cb
                b = cb % NBUF
                nb = (cb + LOOKAHEAD) % NBUF
                wait_gather(b)
                scale(b)
                start_scatter(ci, b)
                wait_scatter(nb)
                start_gather(ci + LOOKAHEAD, nb)
            return c

        lax.fori_loop(0, (n_chunks - 2 * LOOKAHEAD) // NBUF, ring_step, 0)

        for ci in range(n_chunks - LOOKAHEAD, n_chunks):
            b = ci % NBUF
            wait_gather(b)
            scale(b)
            start_scatter(ci, b)
        for b in range(NBUF):
            wait_scatter(b)

    return k


def kernel(x, table):
    S, T = x.shape
    B = S * T
    flat = x.reshape(B).astype(jnp.int32)
    out = _embed_kernel(B)(flat, table)
    return out.reshape(S, T, DIM)
